# baseline (speedup 1.0000x reference)
import functools

import jax
import jax.numpy as jnp
from jax import lax
from jax.experimental import pallas as pl
from jax.experimental.pallas import tpu as pltpu
from jax.experimental.pallas import tpu_sc as plsc

_NUM_CORES = 2
_NUM_SUBCORES = 16
_NW = _NUM_CORES * _NUM_SUBCORES
_LEAK = 0.01


def _lrelu(x):
    return jnp.where(x > 0, x, _LEAK * x)


def _mesh():
    return plsc.VectorSubcoreMesh(
        core_axis_name="c", subcore_axis_name="s",
        num_cores=_NUM_CORES, num_subcores=_NUM_SUBCORES)


def _wid():
    return lax.axis_index("s") * _NUM_CORES + lax.axis_index("c")


_NB = 8


def _make_gather(d, n_idx_rows):
    assert n_idx_rows % (_NW * _NB) == 0
    rows_per_w = n_idx_rows // _NW
    n_rounds = rows_per_w // _NB

    @functools.partial(
        pl.kernel,
        out_type=jax.ShapeDtypeStruct((n_idx_rows * 128, d), jnp.float32),
        mesh=_mesh(),
        scratch_types=[
            pltpu.VMEM((rows_per_w, 128), jnp.int32),
            pltpu.VMEM((_NB * 128, d), jnp.float32),
            pltpu.SemaphoreType.DMA,
            pltpu.SemaphoreType.DMA,
        ],
        compiler_params=pltpu.CompilerParams(use_tc_tiling_on_sc=False),
    )
    def gather_kernel(tab_hbm, idx_hbm, out_hbm, idx_v, rows_v, g_sem, st_sem):
        base = _wid() * rows_per_w
        pltpu.sync_copy(idx_hbm.at[pl.ds(base, rows_per_w)], idx_v)

        def round_body(t, carry):
            @pl.when(t > 0)
            def _():
                pltpu.make_async_copy(
                    rows_v, out_hbm.at[pl.ds(0, _NB * 128)], st_sem).wait()

            for s in range(_NB):
                pltpu.async_copy(
                    tab_hbm.at[idx_v.at[t * _NB + s]],
                    rows_v.at[pl.ds(s * 128, 128)],
                    g_sem)
            pltpu.make_async_copy(
                tab_hbm.at[pl.ds(0, _NB * 128)], rows_v, g_sem).wait()
            pltpu.async_copy(
                rows_v,
                out_hbm.at[pl.ds((base + t * _NB) * 128, _NB * 128)],
                st_sem)
            return carry

        lax.fori_loop(0, n_rounds, round_body, 0)
        pltpu.make_async_copy(
            rows_v, out_hbm.at[pl.ds(0, _NB * 128)], st_sem).wait()

    return gather_kernel


def _make_mlp(n_pairs, blk):
    assert n_pairs % blk == 0 and blk % 128 == 0
    grid = (n_pairs // blk,)

    def mlp_body(g_ref, w1_ref, b1_ref, w2_ref, b2_ref, w3_ref, b3_ref, v_ref):
        g = g_ref[...]
        h1 = _lrelu(
            lax.dot_general(g, w1_ref[...], (((1,), (1,)), ((), ())),
                            preferred_element_type=jnp.float32)
            + b1_ref[...])
        h2 = _lrelu(
            lax.dot_general(h1, w2_ref[...], (((1,), (1,)), ((), ())),
                            preferred_element_type=jnp.float32)
            + b2_ref[...])
        v = _lrelu(jnp.sum(h2 * w3_ref[...], axis=1) + b3_ref[0, 0])
        v_ref[...] = v.reshape(blk // 128, 128)

    return pl.pallas_call(
        mlp_body,
        grid=grid,
        in_specs=[
            pl.BlockSpec((blk, 128), lambda i: (i, 0)),
            pl.BlockSpec((64, 128), lambda i: (0, 0)),
            pl.BlockSpec((1, 64), lambda i: (0, 0)),
            pl.BlockSpec((32, 64), lambda i: (0, 0)),
            pl.BlockSpec((1, 32), lambda i: (0, 0)),
            pl.BlockSpec((1, 32), lambda i: (0, 0)),
            pl.BlockSpec(memory_space=pltpu.SMEM),
        ],
        out_specs=pl.BlockSpec((blk // 128, 128), lambda i: (i, 0)),
        out_shape=jax.ShapeDtypeStruct((n_pairs // 128, 128), jnp.float32),
    )


def _make_scatter(n_minor, n_major, n_pair_rows, pb):
    assert n_major % (2 * pb) == 0
    half = n_major // 2
    n_bands = half // pb
    band_el = pb * n_minor
    rows_per_t = n_pair_rows // _NUM_SUBCORES
    cap_rows = rows_per_t + 1

    @functools.partial(
        pl.kernel,
        out_type=jax.ShapeDtypeStruct((n_major * n_minor,), jnp.float32),
        mesh=_mesh(),
        scratch_types=[
            pltpu.VMEM((rows_per_t, 128), jnp.int32),
            pltpu.VMEM((rows_per_t, 128), jnp.int32),
            pltpu.VMEM((rows_per_t, 128), jnp.float32),
            pltpu.VMEM((cap_rows * 128,), jnp.int32),
            pltpu.VMEM((cap_rows * 128,), jnp.float32),
            pltpu.VMEM((cap_rows, 128), jnp.int32),
            pltpu.VMEM((cap_rows, 128), jnp.float32),
            pltpu.VMEM_SHARED((band_el + 128,), jnp.float32),
            pltpu.SemaphoreType.DMA,
        ],
        compiler_params=pltpu.CompilerParams(
            use_tc_tiling_on_sc=False, needs_layout_passes=False),
    )
    def scatter_kernel(i0_hbm, i1_hbm, v_hbm, z_hbm, out_hbm,
                       i0_v, i1_v, v_v, lidx1, val1, lidx2, val2, band_sp,
                       sc_sem):
        kc = lax.axis_index("c")
        ks = lax.axis_index("s")
        base = ks * rows_per_t
        pltpu.sync_copy(i0_hbm.at[pl.ds(base, rows_per_t)], i0_v)
        pltpu.sync_copy(i1_hbm.at[pl.ds(base, rows_per_t)], i1_v)
        pltpu.sync_copy(v_hbm.at[pl.ds(base, rows_per_t)], v_v)

        dump = jnp.full((16,), band_el, jnp.int32)

        def band_body(b, carry):
            pabs0 = kc * half + b * pb

            @pl.when(ks < 10)
            def _():
                pltpu.sync_copy(z_hbm.at[pl.ds(0, band_el // 10)],
                                band_sp.at[pl.ds(ks * (band_el // 10),
                                                 band_el // 10)])
            plsc.subcore_barrier()

            def scan_row(r, cnt):
                ps = [i1_v[r, pl.ds(k * 16, 16)] for k in range(8)]
                ms = [(p >= pabs0) & (p < pabs0 + pb) for p in ps]
                any_m = ms[0]
                for k in range(1, 8):
                    any_m = any_m | ms[k]
                n_row = plsc.all_reduce_population_count(any_m)[0]

                def hot(cnt2):
                    for k in range(8):
                        p, m = ps[k], ms[k]
                        c = i0_v[r, pl.ds(k * 16, 16)]
                        val = v_v[r, pl.ds(k * 16, 16)]
                        lidx = (p - pabs0) * n_minor + c
                        mi = m.astype(jnp.int32)
                        pos = cnt2 + plsc.cumsum(mi) - mi
                        plsc.store_scatter(lidx1, [pos], lidx, mask=m)
                        plsc.store_scatter(val1, [pos], val, mask=m)
                        cnt2 = cnt2 + plsc.all_reduce_population_count(m)[0]
                    return cnt2

                return lax.cond(n_row > 0, hot, lambda cnt2: cnt2, cnt)

            cnt = lax.fori_loop(0, rows_per_t, scan_row, 0)
            n_rows = (cnt + 127) >> 7

            def padfill(q, c2):
                lidx1[pl.ds(cnt + q * 16, 16)] = dump
                return c2
            lax.fori_loop(0, 8, padfill, 0)

            def to2d(i, c2):
                j = i >> 3
                k = i & 7
                lidx2[j, pl.ds(k * 16, 16)] = lidx1[pl.ds(i * 16, 16)]
                val2[j, pl.ds(k * 16, 16)] = val1[pl.ds(i * 16, 16)]
                return c2
            lax.fori_loop(0, n_rows * 8, to2d, 0)

            def scat_row(j, c2):
                pltpu.async_copy(val2.at[j], band_sp.at[lidx2.at[j]], sc_sem)
                return c2
            lax.fori_loop(0, n_rows, scat_row, 0)

            def scat_drain(j, c2):
                pltpu.make_async_copy(
                    z_hbm.at[pl.ds(0, 128)], val2.at[j], sc_sem).wait()
                return c2
            lax.fori_loop(0, n_rows, scat_drain, 0)
            plsc.subcore_barrier()

            @pl.when(ks == 0)
            def _():
                pltpu.sync_copy(band_sp.at[pl.ds(0, band_el)],
                                out_hbm.at[pl.ds(pabs0 * n_minor, band_el)])
            plsc.subcore_barrier()
            return carry

        lax.fori_loop(0, n_bands, band_body, 0)

    return scatter_kernel


def kernel(x0, x1, W1, b1, W2, b2, W3, b3, idx0, idx1):
    n_comp, d0 = x0.shape
    n_prot, d1 = x1.shape
    b = idx0.shape[0]
    bp = ((b + 32767) // 32768) * 32768
    pad = bp - b

    idx0p = jnp.concatenate([idx0, jnp.broadcast_to(idx0[:1], (pad,))])
    idx1p = jnp.concatenate([idx1, jnp.broadcast_to(idx1[:1], (pad,))])

    tab = jnp.concatenate([x0, x1], axis=0)
    ii = jnp.stack([idx0p, idx1p + n_comp], axis=1).reshape(2 * bp // 128, 128)

    g = _make_gather(d0, 2 * bp // 128)(tab, ii)
    g = g.reshape(bp, d0 + d1)

    v2d = _make_mlp(bp, 2048)(
        g, W1, b1.reshape(1, -1), W2, b2.reshape(1, -1),
        W3.reshape(1, -1), b3.reshape(1, 1))

    out_flat = _make_scatter(n_comp, n_prot, bp // 128, 25)(
        idx0p.reshape(bp // 128, 128),
        idx1p.reshape(bp // 128, 128),
        v2d,
        jnp.zeros((25 * n_comp,), jnp.float32))
    return out_flat.reshape(n_prot, n_comp).T


# speedup vs baseline: 1.0134x; 1.0134x over previous
import functools

import jax
import jax.numpy as jnp
from jax import lax
from jax.experimental import pallas as pl
from jax.experimental.pallas import tpu as pltpu
from jax.experimental.pallas import tpu_sc as plsc

_NUM_CORES = 2
_NUM_SUBCORES = 16
_NW = _NUM_CORES * _NUM_SUBCORES
_LEAK = 0.01


def _lrelu(x):
    return jnp.where(x > 0, x, _LEAK * x)


def _mesh():
    return plsc.VectorSubcoreMesh(
        core_axis_name="c", subcore_axis_name="s",
        num_cores=_NUM_CORES, num_subcores=_NUM_SUBCORES)


def _wid():
    return lax.axis_index("s") * _NUM_CORES + lax.axis_index("c")


_NB = 8


def _make_gather(d, n_idx_rows):
    assert n_idx_rows % (_NW * _NB) == 0
    rows_per_w = n_idx_rows // _NW
    n_rounds = rows_per_w // _NB

    @functools.partial(
        pl.kernel,
        out_type=jax.ShapeDtypeStruct((n_idx_rows * 128, d), jnp.float32),
        mesh=_mesh(),
        scratch_types=[
            pltpu.VMEM((rows_per_w, 128), jnp.int32),
            pltpu.VMEM((_NB * 128, d), jnp.float32),
            pltpu.SemaphoreType.DMA,
            pltpu.SemaphoreType.DMA,
        ],
        compiler_params=pltpu.CompilerParams(use_tc_tiling_on_sc=False),
    )
    def gather_kernel(tab_hbm, idx_hbm, out_hbm, idx_v, rows_v, g_sem, st_sem):
        base = _wid() * rows_per_w
        pltpu.sync_copy(idx_hbm.at[pl.ds(base, rows_per_w)], idx_v)

        def round_body(t, carry):
            @pl.when(t > 0)
            def _():
                pltpu.make_async_copy(
                    rows_v, out_hbm.at[pl.ds(0, _NB * 128)], st_sem).wait()

            for s in range(_NB):
                pltpu.async_copy(
                    tab_hbm.at[idx_v.at[t * _NB + s]],
                    rows_v.at[pl.ds(s * 128, 128)],
                    g_sem)
            pltpu.make_async_copy(
                tab_hbm.at[pl.ds(0, _NB * 128)], rows_v, g_sem).wait()
            pltpu.async_copy(
                rows_v,
                out_hbm.at[pl.ds((base + t * _NB) * 128, _NB * 128)],
                st_sem)
            return carry

        lax.fori_loop(0, n_rounds, round_body, 0)
        pltpu.make_async_copy(
            rows_v, out_hbm.at[pl.ds(0, _NB * 128)], st_sem).wait()

    return gather_kernel


def _make_mlp(n_pairs, blk):
    assert n_pairs % blk == 0 and blk % 128 == 0
    grid = (n_pairs // blk,)

    def mlp_body(g_ref, w1_ref, b1_ref, w2_ref, b2_ref, w3_ref, b3_ref, v_ref):
        g = g_ref[...]
        h1 = _lrelu(
            lax.dot_general(g, w1_ref[...], (((1,), (1,)), ((), ())),
                            preferred_element_type=jnp.float32)
            + b1_ref[...])
        h2 = _lrelu(
            lax.dot_general(h1, w2_ref[...], (((1,), (1,)), ((), ())),
                            preferred_element_type=jnp.float32)
            + b2_ref[...])
        v = _lrelu(jnp.sum(h2 * w3_ref[...], axis=1) + b3_ref[0, 0])
        v_ref[...] = v.reshape(blk // 128, 128)

    return pl.pallas_call(
        mlp_body,
        grid=grid,
        in_specs=[
            pl.BlockSpec((blk, 128), lambda i: (i, 0)),
            pl.BlockSpec((64, 128), lambda i: (0, 0)),
            pl.BlockSpec((1, 64), lambda i: (0, 0)),
            pl.BlockSpec((32, 64), lambda i: (0, 0)),
            pl.BlockSpec((1, 32), lambda i: (0, 0)),
            pl.BlockSpec((1, 32), lambda i: (0, 0)),
            pl.BlockSpec(memory_space=pltpu.SMEM),
        ],
        out_specs=pl.BlockSpec((blk // 128, 128), lambda i: (i, 0)),
        out_shape=jax.ShapeDtypeStruct((n_pairs // 128, 128), jnp.float32),
    )


def _make_scatter(n_minor, n_major, n_pair_rows, pb):
    assert n_major % (2 * pb) == 0
    half = n_major // 2
    n_bands = half // pb
    band_el = pb * n_minor
    rows_per_t = n_pair_rows // _NUM_SUBCORES
    cap_rows = rows_per_t + 1

    @functools.partial(
        pl.kernel,
        out_type=jax.ShapeDtypeStruct((n_major * n_minor,), jnp.float32),
        mesh=_mesh(),
        scratch_types=[
            pltpu.VMEM((rows_per_t, 128), jnp.int32),
            pltpu.VMEM((rows_per_t, 128), jnp.int32),
            pltpu.VMEM((rows_per_t, 128), jnp.float32),
            pltpu.VMEM((cap_rows * 128,), jnp.int32),
            pltpu.VMEM((cap_rows * 128,), jnp.float32),
            pltpu.VMEM((cap_rows, 128), jnp.int32),
            pltpu.VMEM((cap_rows, 128), jnp.float32),
            pltpu.VMEM_SHARED((band_el + 128,), jnp.float32),
        ],
        compiler_params=pltpu.CompilerParams(
            use_tc_tiling_on_sc=False, needs_layout_passes=False),
    )
    def scatter_kernel(i0_hbm, i1_hbm, v_hbm, z_hbm, out_hbm,
                       i0_v, i1_v, v_v, lidx1, val1, lidx2, val2, band_sp):
        kc = lax.axis_index("c")
        ks = lax.axis_index("s")
        base = ks * rows_per_t
        pltpu.sync_copy(i0_hbm.at[pl.ds(base, rows_per_t)], i0_v)
        pltpu.sync_copy(i1_hbm.at[pl.ds(base, rows_per_t)], i1_v)
        pltpu.sync_copy(v_hbm.at[pl.ds(base, rows_per_t)], v_v)

        dump = jnp.full((16,), band_el, jnp.int32)

        def band_body(b, carry):
            pabs0 = kc * half + b * pb

            @pl.when(ks == 0)
            def _():
                pltpu.sync_copy(z_hbm, band_sp.at[pl.ds(0, band_el)])
            plsc.subcore_barrier()

            def scan_row(r, cnt):
                ps = [i1_v[r, pl.ds(k * 16, 16)] for k in range(8)]
                ms = [(p >= pabs0) & (p < pabs0 + pb) for p in ps]
                any_m = ms[0]
                for k in range(1, 8):
                    any_m = any_m | ms[k]
                n_row = plsc.all_reduce_population_count(any_m)[0]

                def hot(cnt2):
                    for k in range(8):
                        p, m = ps[k], ms[k]
                        c = i0_v[r, pl.ds(k * 16, 16)]
                        val = v_v[r, pl.ds(k * 16, 16)]
                        lidx = (p - pabs0) * n_minor + c
                        mi = m.astype(jnp.int32)
                        pos = cnt2 + plsc.cumsum(mi) - mi
                        plsc.store_scatter(lidx1, [pos], lidx, mask=m)
                        plsc.store_scatter(val1, [pos], val, mask=m)
                        cnt2 = cnt2 + plsc.all_reduce_population_count(m)[0]
                    return cnt2

                return lax.cond(n_row > 0, hot, lambda cnt2: cnt2, cnt)

            cnt = lax.fori_loop(0, rows_per_t, scan_row, 0)
            n_rows = (cnt + 127) >> 7

            def padfill(q, c2):
                lidx1[pl.ds(cnt + q * 16, 16)] = dump
                return c2
            lax.fori_loop(0, 8, padfill, 0)

            def to2d(i, c2):
                j = i >> 3
                k = i & 7
                lidx2[j, pl.ds(k * 16, 16)] = lidx1[pl.ds(i * 16, 16)]
                val2[j, pl.ds(k * 16, 16)] = val1[pl.ds(i * 16, 16)]
                return c2
            lax.fori_loop(0, n_rows * 8, to2d, 0)

            def scat_row(j, c2):
                pltpu.sync_copy(val2.at[j], band_sp.at[lidx2.at[j]])
                return c2
            lax.fori_loop(0, n_rows, scat_row, 0)
            plsc.subcore_barrier()

            @pl.when(ks == 0)
            def _():
                pltpu.sync_copy(band_sp.at[pl.ds(0, band_el)],
                                out_hbm.at[pl.ds(pabs0 * n_minor, band_el)])
            plsc.subcore_barrier()
            return carry

        lax.fori_loop(0, n_bands, band_body, 0)

    return scatter_kernel


def kernel(x0, x1, W1, b1, W2, b2, W3, b3, idx0, idx1):
    n_comp, d0 = x0.shape
    n_prot, d1 = x1.shape
    b = idx0.shape[0]
    bp = ((b + 32767) // 32768) * 32768
    pad = bp - b

    idx0p = jnp.concatenate([idx0, jnp.broadcast_to(idx0[:1], (pad,))])
    idx1p = jnp.concatenate([idx1, jnp.broadcast_to(idx1[:1], (pad,))])

    tab = jnp.concatenate([x0, x1], axis=0)
    ii = jnp.stack([idx0p, idx1p + n_comp], axis=1).reshape(2 * bp // 128, 128)

    g = _make_gather(d0, 2 * bp // 128)(tab, ii)
    g = g.reshape(bp, d0 + d1)

    v2d = _make_mlp(bp, 2048)(
        g, W1, b1.reshape(1, -1), W2, b2.reshape(1, -1),
        W3.reshape(1, -1), b3.reshape(1, 1))

    out_flat = _make_scatter(n_comp, n_prot, bp // 128, 25)(
        idx0p.reshape(bp // 128, 128),
        idx1p.reshape(bp // 128, 128),
        v2d,
        jnp.zeros((25 * n_comp,), jnp.float32))
    return out_flat.reshape(n_prot, n_comp).T
